# baseline (speedup 1.0000x reference)
import jax
import jax.numpy as jnp
from jax.experimental import pallas as pl
from jax.experimental.pallas import tpu as pltpu


def _add_kernel(x_ref, emb_ref, o_ref):
    o_ref[...] = x_ref[...] + emb_ref[...]


def kernel(x, emb_weight):
    batch, seq_len, d_model = x.shape

    s_blk = 512
    while seq_len % s_blk:
        s_blk //= 2
    num_s = seq_len // s_blk

    x2 = x.reshape(batch * seq_len, d_model)
    out = pl.pallas_call(
        _add_kernel,
        grid=(num_s, batch),
        in_specs=[
            pl.BlockSpec((s_blk, d_model), lambda s, b: (b * num_s + s, 0)),
            pl.BlockSpec((s_blk, d_model), lambda s, b: (s, 0)),
        ],
        out_specs=pl.BlockSpec((s_blk, d_model), lambda s, b: (b * num_s + s, 0)),
        out_shape=jax.ShapeDtypeStruct((batch * seq_len, d_model), x.dtype),
        compiler_params=pltpu.CompilerParams(
            dimension_semantics=("arbitrary", "arbitrary"),
        ),
    )(x2, emb_weight)
    return out.reshape(batch, seq_len, d_model)


# speedup vs baseline: 1.1846x; 1.1846x over previous
import jax
import jax.numpy as jnp
from jax.experimental import pallas as pl
from jax.experimental.pallas import tpu as pltpu


def _add_kernel(x_ref, emb_ref, o_ref):
    o_ref[...] = x_ref[...] + emb_ref[...]


def kernel(x, emb_weight):
    batch, seq_len, d_model = x.shape

    s_blk = 2048
    while seq_len % s_blk:
        s_blk //= 2
    num_s = seq_len // s_blk

    x2 = x.reshape(batch * seq_len, d_model)
    out = pl.pallas_call(
        _add_kernel,
        grid=(num_s, batch),
        in_specs=[
            pl.BlockSpec((s_blk, d_model), lambda s, b: (b * num_s + s, 0)),
            pl.BlockSpec((s_blk, d_model), lambda s, b: (s, 0)),
        ],
        out_specs=pl.BlockSpec((s_blk, d_model), lambda s, b: (b * num_s + s, 0)),
        out_shape=jax.ShapeDtypeStruct((batch * seq_len, d_model), x.dtype),
        compiler_params=pltpu.CompilerParams(
            dimension_semantics=("arbitrary", "arbitrary"),
        ),
    )(x2, emb_weight)
    return out.reshape(batch, seq_len, d_model)
